# baseline (speedup 1.0000x reference)
import functools

import jax
import jax.numpy as jnp
from jax import lax
from jax.experimental import pallas as pl
from jax.experimental.pallas import tpu as pltpu
from jax.experimental.pallas import tpu_sc as plsc

_NC = 2
_NS = 16
_NW = _NC * _NS
_L = 16

_IVEC = 128


@functools.lru_cache(maxsize=None)
def _make_repack(v: int, d: int):
    assert d == 64
    nvb_full = v // 128
    v_tail = v - nvb_full * 128
    tail_rows = (v_tail + 15) // 16 * 16
    out_rows = nvb_full * 64 + tail_rows // 2
    n_vb = nvb_full + 1
    per_w = (n_vb + _NW - 1) // _NW

    mesh = plsc.VectorSubcoreMesh(core_axis_name="c", subcore_axis_name="s")

    @functools.partial(
        pl.kernel,
        out_type=jax.ShapeDtypeStruct((out_rows, 2 * d), jnp.float32),
        mesh=mesh,
        compiler_params=pltpu.CompilerParams(needs_layout_passes=False),
        scratch_types=[
            pltpu.VMEM((2, d, 128), jnp.float32),
            pltpu.VMEM((2, d, 128), jnp.float32),
            pltpu.VMEM((tail_rows, 128), jnp.float32),
            pltpu.SemaphoreType.DMA((2,)),
            pltpu.SemaphoreType.DMA((2,)),
            pltpu.SemaphoreType.DMA,
        ],
    )
    def repack_kernel(wt_t, tail_pad, tbl2, vin, vout, vtail,
                      i_sem, o_sem, t_sem):
        wid = lax.axis_index("s") * _NC + lax.axis_index("c")
        lo = wid * per_w
        hi = jnp.minimum(lo + per_w, n_vb)

        iota = lax.iota(jnp.int32, _L)

        def start_in(vb, b):
            pltpu.async_copy(
                wt_t.at[:, pl.ds(vb * 128, 128)], vin.at[b], i_sem.at[b]
            )

        def wait_in(b):
            pltpu.make_async_copy(
                wt_t.at[:, pl.ds(0, 128)], vin.at[b], i_sem.at[b]
            ).wait()

        def wait_out(b):
            pltpu.make_async_copy(
                vout.at[b], tbl2.at[pl.ds(0, d)], o_sem.at[b]
            ).wait()

        @pl.when(lo < n_vb)
        def _():
            start_in(jnp.minimum(lo, nvb_full - 1), 0)
            start_in(jnp.minimum(lo + 1, nvb_full - 1), 1)

            def body(vb, _):
                b = lax.rem(vb - lo, 2)
                wait_in(b)

                @pl.when(vb >= lo + 2)
                def _():
                    wait_out(b)

                @pl.when(vb < nvb_full)
                def _():
                    for vc in range(128):
                        k, c0 = (vc * d) // 128, (vc * d) % 128
                        for dg in range(d // _L):
                            vals = plsc.load_gather(
                                vin.at[b],
                                [dg * _L + iota,
                                 jnp.full((_L,), vc, jnp.int32)],
                            )
                            vout[b, k, pl.ds(c0 + dg * _L, _L)] = vals

                    pltpu.async_copy(
                        vout.at[b],
                        tbl2.at[pl.ds(vb * 64, 64)],
                        o_sem.at[b],
                    )

                @pl.when(vb >= nvb_full)
                def _():
                    pltpu.async_copy(tail_pad, vtail, t_sem).wait()
                    for r in range(tail_rows):
                        for dg in range(d // _L):
                            k, c0 = (r * d) // 128, (r * d) % 128
                            vout[b, k, pl.ds(c0 + dg * _L, _L)] = (
                                vtail[r, pl.ds(dg * _L, _L)]
                            )
                    pltpu.async_copy(
                        vout.at[b, pl.ds(0, tail_rows // 2)],
                        tbl2.at[pl.ds(nvb_full * 64, tail_rows // 2)],
                        o_sem.at[b],
                    )

                @pl.when(vb + 2 < hi)
                def _():
                    start_in(jnp.minimum(vb + 2, nvb_full - 1), b)
                return _

            lax.fori_loop(lo, hi, body, None, unroll=False)

            n_w = hi - lo
            @pl.when(n_w >= 2)
            def _():
                wait_out(lax.rem(hi - lo, 2))
            last_b = lax.rem(hi - 1 - lo, 2)
            @pl.when(hi == n_vb)
            def _():
                pltpu.make_async_copy(
                    vout.at[0, pl.ds(0, tail_rows // 2)],
                    tbl2.at[pl.ds(0, tail_rows // 2)],
                    o_sem.at[last_b],
                ).wait()
            @pl.when(hi < n_vb)
            def _():
                wait_out(last_b)

    return repack_kernel


@functools.lru_cache(maxsize=None)
def _make_gather(n: int, vpad: int, d: int, chunk: int):
    per_w = n // _NW
    n_chunks = per_w // chunk
    u = chunk // _IVEC
    assert per_w % chunk == 0 and chunk % _IVEC == 0

    mesh = plsc.VectorSubcoreMesh(core_axis_name="c", subcore_axis_name="s")

    @functools.partial(
        pl.kernel,
        out_type=jax.ShapeDtypeStruct((n, 2 * d), jnp.float32),
        mesh=mesh,
        compiler_params=pltpu.CompilerParams(use_tc_tiling_on_sc=False),
        scratch_types=[
            pltpu.VMEM((4, chunk), jnp.int32),
            pltpu.VMEM((3, chunk, d), jnp.float32),
            pltpu.SemaphoreType.DMA((4,)),
            pltpu.SemaphoreType.DMA((3,)),
            pltpu.SemaphoreType.DMA((3,)),
        ],
    )
    def gather_kernel(idx_hbm, table_hbm, out_hbm, idx_v, rows_v,
                      idx_sem, g_sem, o_sem):
        wid = lax.axis_index("s") * _NC + lax.axis_index("c")
        base = wid * per_w

        def start_idx_copy(g, s):
            pltpu.async_copy(
                idx_hbm.at[pl.ds(base + g * chunk, chunk)],
                idx_v.at[s],
                idx_sem.at[s],
            )

        def fire_gathers(g, s, b):
            for j in range(u):
                pltpu.async_copy(
                    table_hbm.at[idx_v.at[s, pl.ds(j * _IVEC, _IVEC)]],
                    rows_v.at[b, pl.ds(j * _IVEC, _IVEC)],
                    g_sem.at[b],
                )

        def drain_gathers(b):
            pltpu.make_async_copy(
                rows_v.at[b], out_hbm.at[pl.ds(0, chunk), pl.ds(0, d)],
                g_sem.at[b],
            ).wait()

        def start_out_copy(g, b):
            pltpu.async_copy(
                rows_v.at[b],
                out_hbm.at[pl.ds(base + g * chunk, chunk), pl.ds(0, d)],
                o_sem.at[b],
            )

        def wait_out_copy(b):
            pltpu.make_async_copy(
                rows_v.at[b], out_hbm.at[pl.ds(0, chunk), pl.ds(0, d)],
                o_sem.at[b],
            ).wait()

        for g in range(4):
            start_idx_copy(g, g)

        def body(g, _):
            s = lax.rem(g, 4)
            b = lax.rem(g, 3)
            pltpu.make_async_copy(
                idx_hbm.at[pl.ds(0, chunk)], idx_v.at[s], idx_sem.at[s]
            ).wait()
            @pl.when(g >= 3)
            def _():
                wait_out_copy(b)
            fire_gathers(g, s, b)
            @pl.when(g >= 1)
            def _():
                bp = lax.rem(g + 2, 3)
                sp = lax.rem(g + 3, 4)
                drain_gathers(bp)
                @pl.when(g + 3 < n_chunks)
                def _():
                    start_idx_copy(g + 3, sp)
                start_out_copy(g - 1, bp)
            return _

        lax.fori_loop(0, n_chunks, body, None, unroll=False)

        bl = lax.rem(n_chunks - 1, 3)
        drain_gathers(bl)
        start_out_copy(n_chunks - 1, bl)
        for b in range(3):
            wait_out_copy(b)

    return gather_kernel


def kernel(word_input, weight_all):
    b, l = word_input.shape
    v, d = weight_all.shape
    n = b * l
    idx = word_input.reshape(n)
    nvb_full = v // 128
    v_main = nvb_full * 128
    tail_rows = (v - v_main + 15) // 16 * 16
    tail_pad = jnp.pad(weight_all[v_main:], ((0, tail_rows - (v - v_main)),
                                             (0, 128 - d)))
    tbl2 = _make_repack(v, d)(weight_all.T, tail_pad)
    tbl = tbl2.reshape(v_main + tail_rows, d)
    out2 = _make_gather(n, v_main + tail_rows, d, 640)(idx, tbl)
    return out2[:, :d].reshape(b, l, d)


# speedup vs baseline: 2.3060x; 2.3060x over previous
import functools

import jax
import jax.numpy as jnp
from jax import lax
from jax.experimental import pallas as pl
from jax.experimental.pallas import tpu as pltpu
from jax.experimental.pallas import tpu_sc as plsc

_NC = 2
_NS = 16
_NW = _NC * _NS

_IVEC = 128
_TB = 2048


@functools.lru_cache(maxsize=None)
def _make_repack(v: int, d: int):
    assert d == 64
    grid = (v + _TB - 1) // _TB
    out_rows = grid * _TB // 2

    def body(in_ref, out_ref):
        a = jnp.transpose(in_ref[:, : _TB // 2], (1, 0))
        c = jnp.transpose(in_ref[:, _TB // 2 :], (1, 0))
        out_ref[...] = jnp.concatenate([a, c], axis=1)

    return pl.pallas_call(
        body,
        grid=(grid,),
        in_specs=[pl.BlockSpec((d, _TB), lambda i: (0, i))],
        out_specs=pl.BlockSpec((_TB // 2, 2 * d), lambda i: (i, 0)),
        out_shape=jax.ShapeDtypeStruct((out_rows, 2 * d), jnp.float32),
    )


@functools.lru_cache(maxsize=None)
def _make_gather(n: int, vpad: int, d: int, chunk: int):
    per_w = n // _NW
    n_chunks = per_w // chunk
    u = chunk // _IVEC
    assert per_w % chunk == 0 and chunk % _IVEC == 0

    mesh = plsc.VectorSubcoreMesh(core_axis_name="c", subcore_axis_name="s")

    @functools.partial(
        pl.kernel,
        out_type=jax.ShapeDtypeStruct((n, 2 * d), jnp.float32),
        mesh=mesh,
        compiler_params=pltpu.CompilerParams(use_tc_tiling_on_sc=False),
        scratch_types=[
            pltpu.VMEM((4, chunk), jnp.int32),
            pltpu.VMEM((3, chunk, d), jnp.float32),
            pltpu.SemaphoreType.DMA((4,)),
            pltpu.SemaphoreType.DMA((3,)),
            pltpu.SemaphoreType.DMA((3,)),
        ],
    )
    def gather_kernel(idx_hbm, table_hbm, out_hbm, idx_v, rows_v,
                      idx_sem, g_sem, o_sem):
        wid = lax.axis_index("s") * _NC + lax.axis_index("c")
        base = wid * per_w

        def start_idx_copy(g, s):
            pltpu.async_copy(
                idx_hbm.at[pl.ds(base + g * chunk, chunk)],
                idx_v.at[s],
                idx_sem.at[s],
            )

        def fire_gathers(g, s, b):
            for j in range(u):
                pltpu.async_copy(
                    table_hbm.at[idx_v.at[s, pl.ds(j * _IVEC, _IVEC)]],
                    rows_v.at[b, pl.ds(j * _IVEC, _IVEC)],
                    g_sem.at[b],
                )

        def drain_gathers(b):
            pltpu.make_async_copy(
                rows_v.at[b], out_hbm.at[pl.ds(0, chunk), pl.ds(0, d)],
                g_sem.at[b],
            ).wait()

        def start_out_copy(g, b):
            pltpu.async_copy(
                rows_v.at[b],
                out_hbm.at[pl.ds(base + g * chunk, chunk), pl.ds(0, d)],
                o_sem.at[b],
            )

        def wait_out_copy(b):
            pltpu.make_async_copy(
                rows_v.at[b], out_hbm.at[pl.ds(0, chunk), pl.ds(0, d)],
                o_sem.at[b],
            ).wait()

        for g in range(4):
            start_idx_copy(g, g)

        def body(g, _):
            s = lax.rem(g, 4)
            b = lax.rem(g, 3)
            pltpu.make_async_copy(
                idx_hbm.at[pl.ds(0, chunk)], idx_v.at[s], idx_sem.at[s]
            ).wait()
            @pl.when(g >= 3)
            def _():
                wait_out_copy(b)
            fire_gathers(g, s, b)
            @pl.when(g >= 1)
            def _():
                bp = lax.rem(g + 2, 3)
                sp = lax.rem(g + 3, 4)
                drain_gathers(bp)
                @pl.when(g + 3 < n_chunks)
                def _():
                    start_idx_copy(g + 3, sp)
                start_out_copy(g - 1, bp)
            return _

        lax.fori_loop(0, n_chunks, body, None, unroll=False)

        bl = lax.rem(n_chunks - 1, 3)
        drain_gathers(bl)
        start_out_copy(n_chunks - 1, bl)
        for b in range(3):
            wait_out_copy(b)

    return gather_kernel


def kernel(word_input, weight_all):
    b, l = word_input.shape
    v, d = weight_all.shape
    n = b * l
    idx = word_input.reshape(n)
    h = _TB // 2
    j = idx % _TB
    idx_r = 2 * ((idx // _TB) * h + j % h) + j // h
    tbl2 = _make_repack(v, d)(weight_all.T)
    vpad = tbl2.shape[0] * 2
    tbl = tbl2.reshape(vpad, d)
    out2 = _make_gather(n, vpad, d, 640)(idx_r, tbl)
    return out2[:, :d].reshape(b, l, d)
